# baseline (speedup 1.0000x reference)
import functools

import jax
import jax.numpy as jnp
from jax import lax
from jax.experimental import pallas as pl
from jax.experimental.pallas import tpu as pltpu
from jax.experimental.pallas import tpu_sc as plsc

_NUM_CORES = 2
_NUM_SUBCORES = 16
_NW = _NUM_CORES * _NUM_SUBCORES

_EPS = 1e-5


def _table_body(tok_ref, pos_ref, g_ref, b_ref, out_ref):
    pos = pos_ref[...]
    g = g_ref[...]
    b = b_ref[...]
    vocab = tok_ref.shape[0]
    rows = []
    for v in range(vocab):
        e = pos + tok_ref[v:v + 1, :]
        m = jnp.mean(e, axis=1, keepdims=True)
        c = e - m
        var = jnp.mean(c * c, axis=1, keepdims=True)
        rows.append(c * lax.rsqrt(var + _EPS) * g + b)
    out_ref[...] = jnp.concatenate(rows, axis=0)


def _table_call(tok, pos, gamma, beta):
    vocab, d = tok.shape
    l = pos.shape[0]
    return pl.pallas_call(
        _table_body,
        out_shape=jax.ShapeDtypeStruct((vocab * l, d), jnp.float32),
    )(tok, pos, gamma.reshape(1, d), beta.reshape(1, d))


@functools.cache
def _gather_call(nb: int, l: int, d: int, vocab: int):
    bw = nb // _NW
    assert bw * _NW == nb and bw % 128 == 0
    ntab = vocab * l * d

    mesh = plsc.VectorSubcoreMesh(core_axis_name="c", subcore_axis_name="s")

    @functools.partial(
        pl.kernel,
        mesh=mesh,
        compiler_params=pltpu.CompilerParams(
            use_tc_tiling_on_sc=True, needs_layout_passes=False),
        out_type=jax.ShapeDtypeStruct((l, d, nb), jnp.float32),
        scratch_types=[
            pltpu.VMEM((ntab,), jnp.float32),
            pltpu.VMEM((2, bw), jnp.int32),
            pltpu.VMEM((2, d, bw), jnp.float32),
            pltpu.SemaphoreType.DMA,
            [pltpu.SemaphoreType.DMA] * 2,
            [pltpu.SemaphoreType.DMA] * 2,
        ],
    )
    def gather(tab_hbm, xt_hbm, out_hbm, tab_v, x_v, blk_v,
               tab_sem, x_sems, o_sems):
        wid = lax.axis_index("s") * _NUM_CORES + lax.axis_index("c")
        b0 = wid * bw

        tab_cp = pltpu.async_copy(tab_hbm, tab_v, tab_sem)

        def x_fetch(li, s):
            return pltpu.async_copy(
                xt_hbm.at[pl.ds(li * nb + b0, bw)], x_v.at[s], x_sems[s])

        def drain_x(s):
            pltpu.make_async_copy(
                xt_hbm.at[pl.ds(0, bw)], x_v.at[s], x_sems[s]).wait()

        def compute(li, s):
            off = li * d

            def bg_body(bg, carry):
                xv = x_v[s, pl.ds(bg * 16, 16)]
                base = xv * (l * d) + off
                for dd in range(d):
                    blk_v[s, dd, pl.ds(bg * 16, 16)] = plsc.load_gather(
                        tab_v, [base + dd])
                return carry

            lax.fori_loop(0, bw // 16, bg_body, None, unroll=False)

        def out_write(li, s):
            return pltpu.async_copy(
                blk_v.at[s], out_hbm.at[li, :, pl.ds(b0, bw)], o_sems[s])

        def drain_out(s):
            pltpu.make_async_copy(
                blk_v.at[s], out_hbm.at[0, :, pl.ds(b0, bw)],
                o_sems[s]).wait()

        x_fetch(0, 0)
        x_fetch(1, 1)
        tab_cp.wait()
        for s in (0, 1):
            drain_x(s)
            compute(s, s)
            out_write(s, s)
            x_fetch(s + 2, s)

        def body2(i2, carry):
            for s in (0, 1):
                li = 2 + i2 * 2 + s
                drain_x(s)
                drain_out(s)
                compute(li, s)
                out_write(li, s)
                x_fetch(li + 2, s)
            return carry

        lax.fori_loop(0, (l - 4) // 2, body2, None, unroll=False)
        for k in range(2):
            li = l - 2 + k
            s = li % 2
            drain_x(s)
            drain_out(s)
            compute(li, s)
            out_write(li, s)
        drain_out(0)
        drain_out(1)

    return gather


def kernel(x, tok_embed, pos_embed, gamma, beta):
    nb, l = x.shape
    vocab, d = tok_embed.shape
    tab = _table_call(tok_embed, pos_embed, gamma, beta).reshape(-1)
    out_t = _gather_call(nb, l, d, vocab)(tab, x.T.reshape(-1))
    return jnp.transpose(out_t, (2, 0, 1))


# speedup vs baseline: 1.3834x; 1.3834x over previous
import functools

import jax
import jax.numpy as jnp
from jax import lax
from jax.experimental import pallas as pl
from jax.experimental.pallas import tpu as pltpu
from jax.experimental.pallas import tpu_sc as plsc

_NUM_CORES = 2
_NUM_SUBCORES = 16
_NW = _NUM_CORES * _NUM_SUBCORES

_EPS = 1e-5


def _table_body(tok_ref, pos_ref, g_ref, b_ref, out_ref):
    pos = pos_ref[...]
    g = g_ref[...]
    b = b_ref[...]
    vocab = tok_ref.shape[0]
    rows = []
    for v in range(vocab):
        e = pos + tok_ref[v:v + 1, :]
        m = jnp.mean(e, axis=1, keepdims=True)
        c = e - m
        var = jnp.mean(c * c, axis=1, keepdims=True)
        rows.append(c * lax.rsqrt(var + _EPS) * g + b)
    out_ref[...] = jnp.concatenate(rows, axis=0)


def _table_call(tok, pos, gamma, beta):
    vocab, d = tok.shape
    l = pos.shape[0]
    return pl.pallas_call(
        _table_body,
        out_shape=jax.ShapeDtypeStruct((vocab * l, d), jnp.float32),
    )(tok, pos, gamma.reshape(1, d), beta.reshape(1, d))


@functools.cache
def _gather_call(nb: int, l: int, d: int, vocab: int):
    bw = nb // _NW
    assert bw * _NW == nb and bw % 128 == 0
    ntab = vocab * l * d

    mesh = plsc.VectorSubcoreMesh(core_axis_name="c", subcore_axis_name="s")

    @functools.partial(
        pl.kernel,
        mesh=mesh,
        compiler_params=pltpu.CompilerParams(
            use_tc_tiling_on_sc=True, needs_layout_passes=False),
        out_type=jax.ShapeDtypeStruct((l, d, nb), jnp.float32),
        scratch_types=[
            pltpu.VMEM((ntab,), jnp.float32),
            pltpu.VMEM((2, bw), jnp.int32),
            pltpu.VMEM((2, d, bw), jnp.float32),
            pltpu.SemaphoreType.DMA,
            [pltpu.SemaphoreType.DMA] * 2,
            [pltpu.SemaphoreType.DMA] * 2,
        ],
    )
    def gather(tab_hbm, xt_hbm, out_hbm, tab_v, x_v, blk_v,
               tab_sem, x_sems, o_sems):
        wid = lax.axis_index("s") * _NUM_CORES + lax.axis_index("c")
        b0 = wid * bw

        tab_cp = pltpu.async_copy(tab_hbm, tab_v, tab_sem)

        def x_fetch(li, s):
            return pltpu.async_copy(
                xt_hbm.at[pl.ds(li * nb + b0, bw)], x_v.at[s], x_sems[s])

        def drain_x(s):
            pltpu.make_async_copy(
                xt_hbm.at[pl.ds(0, bw)], x_v.at[s], x_sems[s]).wait()

        def compute(li, s):
            off = li * d

            @plsc.parallel_loop(0, bw // 16, unroll=4)
            def _bg_body(bg):
                xv = x_v[s, pl.ds(bg * 16, 16)]
                base = xv * (l * d) + off
                for dd in range(d):
                    blk_v[s, dd, pl.ds(bg * 16, 16)] = plsc.load_gather(
                        tab_v, [base + dd])

        def out_write(li, s):
            return pltpu.async_copy(
                blk_v.at[s], out_hbm.at[li, :, pl.ds(b0, bw)], o_sems[s])

        def drain_out(s):
            pltpu.make_async_copy(
                blk_v.at[s], out_hbm.at[0, :, pl.ds(b0, bw)],
                o_sems[s]).wait()

        x_fetch(0, 0)
        x_fetch(1, 1)
        tab_cp.wait()
        for s in (0, 1):
            drain_x(s)
            compute(s, s)
            out_write(s, s)
            x_fetch(s + 2, s)

        def body2(i2, carry):
            for s in (0, 1):
                li = 2 + i2 * 2 + s
                drain_x(s)
                drain_out(s)
                compute(li, s)
                out_write(li, s)
                x_fetch(li + 2, s)
            return carry

        lax.fori_loop(0, (l - 4) // 2, body2, None, unroll=False)
        for k in range(2):
            li = l - 2 + k
            s = li % 2
            drain_x(s)
            drain_out(s)
            compute(li, s)
            out_write(li, s)
        drain_out(0)
        drain_out(1)

    return gather


def kernel(x, tok_embed, pos_embed, gamma, beta):
    nb, l = x.shape
    vocab, d = tok_embed.shape
    tab = _table_call(tok_embed, pos_embed, gamma, beta).reshape(-1)
    out_t = _gather_call(nb, l, d, vocab)(tab, x.T.reshape(-1))
    return jnp.transpose(out_t, (2, 0, 1))


# speedup vs baseline: 6.7425x; 4.8739x over previous
import functools

import jax
import jax.numpy as jnp
from jax import lax
from jax.experimental import pallas as pl
from jax.experimental.pallas import tpu as pltpu
from jax.experimental.pallas import tpu_sc as plsc

_NUM_CORES = 2
_NUM_SUBCORES = 16
_NW = _NUM_CORES * _NUM_SUBCORES

_EPS = 1e-5
_L_SC = 8


def _ln_rows(tok_ref, pos, g, b):
    rows = []
    for v in range(tok_ref.shape[0]):
        e = pos + tok_ref[v:v + 1, :]
        m = jnp.mean(e, axis=1, keepdims=True)
        c = e - m
        var = jnp.mean(c * c, axis=1, keepdims=True)
        rows.append(c * lax.rsqrt(var + _EPS) * g + b)
    return rows


def _table_body(tok_ref, pos_ref, g_ref, b_ref, out_ref):
    out_ref[...] = jnp.concatenate(
        _ln_rows(tok_ref, pos_ref[...], g_ref[...], b_ref[...]), axis=0)


def _table_call(tok, pos, gamma, beta):
    vocab, d = tok.shape
    l = pos.shape[0]
    return pl.pallas_call(
        _table_body,
        out_shape=jax.ShapeDtypeStruct((vocab * l, d), jnp.float32),
    )(tok, pos, gamma.reshape(1, d), beta.reshape(1, d))


def _select_body(xt_ref, tok_ref, pos_ref, g_ref, b_ref, out_ref):
    d = tok_ref.shape[1]
    xt = xt_ref[0]
    acc = None
    for v, row in enumerate(
            _ln_rows(tok_ref, pos_ref[0], g_ref[...], b_ref[...])):
        col = row.reshape(d, 1)
        if acc is None:
            acc = jnp.broadcast_to(col, out_ref.shape[1:])
        else:
            acc = jnp.where(xt == v, col, acc)
    out_ref[...] = acc[None]


@functools.cache
def _tc_select_call(nb: int, l: int, vocab: int, d: int, l_tc: int, bb: int):
    return pl.pallas_call(
        _select_body,
        grid=(l_tc, nb // bb),
        in_specs=[
            pl.BlockSpec((1, 1, bb), lambda i, j: (i, 0, j)),
            pl.BlockSpec((vocab, d), lambda i, j: (0, 0)),
            pl.BlockSpec((1, 1, d), lambda i, j: (i, 0, 0)),
            pl.BlockSpec((1, d), lambda i, j: (0, 0)),
            pl.BlockSpec((1, d), lambda i, j: (0, 0)),
        ],
        out_specs=pl.BlockSpec((1, d, bb), lambda i, j: (i, 0, j)),
        out_shape=jax.ShapeDtypeStruct((l, d, nb), jnp.float32),
    )


@functools.cache
def _sc_gather_call(nb: int, l: int, d: int, vocab: int, l0: int, nl: int):
    bw = nb // _NW
    assert bw * _NW == nb and bw % 128 == 0 and nl >= 4 and nl % 2 == 0
    ntab = vocab * l * d

    mesh = plsc.VectorSubcoreMesh(core_axis_name="c", subcore_axis_name="s")

    @functools.partial(
        pl.kernel,
        mesh=mesh,
        compiler_params=pltpu.CompilerParams(
            use_tc_tiling_on_sc=True, needs_layout_passes=False),
        out_type=jax.ShapeDtypeStruct((nl, d, nb), jnp.float32),
        scratch_types=[
            pltpu.VMEM((ntab,), jnp.float32),
            pltpu.VMEM((2, bw), jnp.int32),
            pltpu.VMEM((2, d, bw), jnp.float32),
            pltpu.SemaphoreType.DMA,
            [pltpu.SemaphoreType.DMA] * 2,
            [pltpu.SemaphoreType.DMA] * 2,
        ],
    )
    def gather(tab_hbm, xt_hbm, out_hbm, tab_v, x_v, blk_v,
               tab_sem, x_sems, o_sems):
        wid = lax.axis_index("s") * _NUM_CORES + lax.axis_index("c")
        b0 = wid * bw

        tab_cp = pltpu.async_copy(tab_hbm, tab_v, tab_sem)

        def x_fetch(li, s):
            return pltpu.async_copy(
                xt_hbm.at[pl.ds((l0 + li) * nb + b0, bw)],
                x_v.at[s], x_sems[s])

        def drain_x(s):
            pltpu.make_async_copy(
                xt_hbm.at[pl.ds(0, bw)], x_v.at[s], x_sems[s]).wait()

        def compute(li, s):
            off = (l0 + li) * d

            @plsc.parallel_loop(0, bw // 16, unroll=4)
            def _bg_body(bg):
                xv = x_v[s, pl.ds(bg * 16, 16)]
                base = xv * (l * d) + off
                for dd in range(d):
                    blk_v[s, dd, pl.ds(bg * 16, 16)] = plsc.load_gather(
                        tab_v, [base + dd])

        def out_write(li, s):
            return pltpu.async_copy(
                blk_v.at[s], out_hbm.at[li, :, pl.ds(b0, bw)], o_sems[s])

        def drain_out(s):
            pltpu.make_async_copy(
                blk_v.at[s], out_hbm.at[0, :, pl.ds(b0, bw)],
                o_sems[s]).wait()

        x_fetch(0, 0)
        x_fetch(1, 1)
        tab_cp.wait()
        for s in (0, 1):
            drain_x(s)
            compute(s, s)
            out_write(s, s)
            x_fetch(s + 2, s)

        def body2(i2, carry):
            for s in (0, 1):
                li = 2 + i2 * 2 + s
                drain_x(s)
                drain_out(s)
                compute(li, s)
                out_write(li, s)
                x_fetch(li + 2, s)
            return carry

        lax.fori_loop(0, (nl - 4) // 2, body2, None, unroll=False)
        for k in range(2):
            li = nl - 2 + k
            s = li % 2
            drain_x(s)
            drain_out(s)
            compute(li, s)
            out_write(li, s)
        drain_out(0)
        drain_out(1)

    return gather


def kernel(x, tok_embed, pos_embed, gamma, beta):
    nb, l = x.shape
    vocab, d = tok_embed.shape
    l_tc = l - _L_SC

    xt3 = x.T.reshape(l, 1, nb)
    tab = _table_call(tok_embed, pos_embed, gamma, beta).reshape(-1)
    sc_part = _sc_gather_call(nb, l, d, vocab, l_tc, _L_SC)(
        tab, x.T.reshape(-1))
    tc_part = _tc_select_call(nb, l, vocab, d, l_tc, 4096)(
        xt3, tok_embed, pos_embed.reshape(l, 1, d),
        gamma.reshape(1, d), beta.reshape(1, d))
    out_t = lax.dynamic_update_slice(tc_part, sc_part, (l_tc, 0, 0))
    return jnp.transpose(out_t, (2, 0, 1))


# speedup vs baseline: 8.9214x; 1.3232x over previous
import functools

import jax
import jax.numpy as jnp
from jax import lax
from jax.experimental import pallas as pl
from jax.experimental.pallas import tpu as pltpu
from jax.experimental.pallas import tpu_sc as plsc

_NUM_CORES = 2
_NUM_SUBCORES = 16
_NW = _NUM_CORES * _NUM_SUBCORES

_EPS = 1e-5
_L_SC = 8


def _ln_rows(tok_ref, pos, g, b):
    rows = []
    for v in range(tok_ref.shape[0]):
        e = pos + tok_ref[v:v + 1, :]
        m = jnp.mean(e, axis=1, keepdims=True)
        c = e - m
        var = jnp.mean(c * c, axis=1, keepdims=True)
        rows.append(c * lax.rsqrt(var + _EPS) * g + b)
    return rows


def _table_body(tok_ref, pos_ref, g_ref, b_ref, out_ref):
    out_ref[...] = jnp.concatenate(
        _ln_rows(tok_ref, pos_ref[...], g_ref[...], b_ref[...]), axis=0)


def _table_call(tok, pos, gamma, beta):
    vocab, d = tok.shape
    l = pos.shape[0]
    return pl.pallas_call(
        _table_body,
        out_shape=jax.ShapeDtypeStruct((vocab * l, d), jnp.float32),
    )(tok, pos, gamma.reshape(1, d), beta.reshape(1, d))


def _select_body(xt_ref, tok_ref, pos_ref, g_ref, b_ref, out_ref):
    d = tok_ref.shape[1]
    xt = xt_ref[0]
    acc = None
    for v, row in enumerate(
            _ln_rows(tok_ref, pos_ref[0], g_ref[...], b_ref[...])):
        col = row.reshape(d, 1)
        if acc is None:
            acc = jnp.broadcast_to(col, out_ref.shape[1:])
        else:
            acc = jnp.where(xt == v, col, acc)
    out_ref[...] = acc[None]


@functools.cache
def _tc_select_call(nb: int, l: int, vocab: int, d: int, l_tc: int, bb: int):
    return pl.pallas_call(
        _select_body,
        grid=(l_tc, nb // bb),
        in_specs=[
            pl.BlockSpec((1, 1, bb), lambda i, j: (i, 0, j)),
            pl.BlockSpec((vocab, d), lambda i, j: (0, 0)),
            pl.BlockSpec((1, 1, d), lambda i, j: (i, 0, 0)),
            pl.BlockSpec((1, d), lambda i, j: (0, 0)),
            pl.BlockSpec((1, d), lambda i, j: (0, 0)),
        ],
        out_specs=pl.BlockSpec((1, d, bb), lambda i, j: (i, 0, j)),
        out_shape=jax.ShapeDtypeStruct((l, d, nb), jnp.float32),
    )


@functools.cache
def _sc_gather_call(nb: int, l: int, d: int, vocab: int, l0: int, nl: int):
    bw = nb // _NW
    assert bw * _NW == nb and bw % 128 == 0 and nl >= 4 and nl % 2 == 0
    ntab = vocab * l * d

    mesh = plsc.VectorSubcoreMesh(core_axis_name="c", subcore_axis_name="s")

    @functools.partial(
        pl.kernel,
        mesh=mesh,
        compiler_params=pltpu.CompilerParams(
            use_tc_tiling_on_sc=True, needs_layout_passes=False),
        out_type=jax.ShapeDtypeStruct((nl, d, nb), jnp.float32),
        scratch_types=[
            pltpu.VMEM((ntab,), jnp.float32),
            pltpu.VMEM((2, bw), jnp.int32),
            pltpu.VMEM((2, d, bw), jnp.float32),
            pltpu.SemaphoreType.DMA,
            [pltpu.SemaphoreType.DMA] * 2,
            [pltpu.SemaphoreType.DMA] * 2,
        ],
    )
    def gather(tab_hbm, xt_hbm, out_hbm, tab_v, x_v, blk_v,
               tab_sem, x_sems, o_sems):
        wid = lax.axis_index("s") * _NUM_CORES + lax.axis_index("c")
        b0 = wid * bw

        tab_cp = pltpu.async_copy(tab_hbm, tab_v, tab_sem)

        def x_fetch(li, s):
            return pltpu.async_copy(
                xt_hbm.at[pl.ds((l0 + li) * nb + b0, bw)],
                x_v.at[s], x_sems[s])

        def drain_x(s):
            pltpu.make_async_copy(
                xt_hbm.at[pl.ds(0, bw)], x_v.at[s], x_sems[s]).wait()

        def compute(li, s):
            off = (l0 + li) * d

            @plsc.parallel_loop(0, bw // 16, unroll=4)
            def _bg_body(bg):
                xv = x_v[s, pl.ds(bg * 16, 16)]
                base = xv * (l * d) + off
                for dd in range(d):
                    blk_v[s, dd, pl.ds(bg * 16, 16)] = plsc.load_gather(
                        tab_v, [base + dd])

        def out_write(li, s):
            return pltpu.async_copy(
                blk_v.at[s], out_hbm.at[li, :, pl.ds(b0, bw)], o_sems[s])

        def drain_out(s):
            pltpu.make_async_copy(
                blk_v.at[s], out_hbm.at[0, :, pl.ds(b0, bw)],
                o_sems[s]).wait()

        x_fetch(0, 0)
        x_fetch(1, 1)
        tab_cp.wait()
        for s in (0, 1):
            drain_x(s)
            compute(s, s)
            out_write(s, s)
            x_fetch(s + 2, s)

        def body2(i2, carry):
            for s in (0, 1):
                li = 2 + i2 * 2 + s
                drain_x(s)
                drain_out(s)
                compute(li, s)
                out_write(li, s)
                x_fetch(li + 2, s)
            return carry

        lax.fori_loop(0, (nl - 4) // 2, body2, None, unroll=False)
        for k in range(2):
            li = nl - 2 + k
            s = li % 2
            drain_x(s)
            drain_out(s)
            compute(li, s)
            out_write(li, s)
        drain_out(0)
        drain_out(1)

    return gather


def kernel(x, tok_embed, pos_embed, gamma, beta):
    nb, l = x.shape
    vocab, d = tok_embed.shape
    l_tc = l - _L_SC

    xt3 = x.T.reshape(l, 1, nb)
    tab = _table_call(tok_embed, pos_embed, gamma, beta).reshape(-1)
    sc_part = _sc_gather_call(nb, l, d, vocab, l_tc, _L_SC)(
        tab, x.T.reshape(-1))
    tc_part = _tc_select_call(nb, l, vocab, d, l_tc, 8192)(
        xt3, tok_embed, pos_embed.reshape(l, 1, d),
        gamma.reshape(1, d), beta.reshape(1, d))
    out_t = lax.dynamic_update_slice(tc_part, sc_part, (l_tc, 0, 0))
    return jnp.transpose(out_t, (2, 0, 1))


# speedup vs baseline: 10.0371x; 1.1251x over previous
import functools

import jax
import jax.numpy as jnp
from jax import lax
from jax.experimental import pallas as pl
from jax.experimental.pallas import tpu as pltpu
from jax.experimental.pallas import tpu_sc as plsc

_NUM_CORES = 2
_NUM_SUBCORES = 16
_NW = _NUM_CORES * _NUM_SUBCORES

_EPS = 1e-5
_L_SC = 8


def _ln_rows(tok_ref, pos, g, b):
    rows = []
    for v in range(tok_ref.shape[0]):
        e = pos + tok_ref[v:v + 1, :]
        m = jnp.mean(e, axis=1, keepdims=True)
        c = e - m
        var = jnp.mean(c * c, axis=1, keepdims=True)
        rows.append(c * lax.rsqrt(var + _EPS) * g + b)
    return rows


def _table_body(tok_ref, pos_ref, g_ref, b_ref, out_ref):
    out_ref[...] = jnp.concatenate(
        _ln_rows(tok_ref, pos_ref[...], g_ref[...], b_ref[...]), axis=0)


def _table_call(tok, pos, gamma, beta):
    vocab, d = tok.shape
    l = pos.shape[0]
    return pl.pallas_call(
        _table_body,
        out_shape=jax.ShapeDtypeStruct((vocab * l, d), jnp.float32),
    )(tok, pos, gamma.reshape(1, d), beta.reshape(1, d))


def _select_body(xt_ref, tok_ref, pos_ref, g_ref, b_ref, out_ref):
    d = tok_ref.shape[1]
    xt = xt_ref[0]
    acc = None
    for v, row in enumerate(
            _ln_rows(tok_ref, pos_ref[0], g_ref[...], b_ref[...])):
        col = row.reshape(d, 1)
        if acc is None:
            acc = jnp.broadcast_to(col, out_ref.shape[1:])
        else:
            acc = jnp.where(xt == v, col, acc)
    out_ref[...] = acc[None]


@functools.cache
def _tc_select_call(nb: int, l: int, vocab: int, d: int, l_tc: int, bb: int):
    return pl.pallas_call(
        _select_body,
        grid=(l_tc, nb // bb),
        in_specs=[
            pl.BlockSpec((1, 1, bb), lambda i, j: (i, 0, j)),
            pl.BlockSpec((vocab, d), lambda i, j: (0, 0)),
            pl.BlockSpec((1, 1, d), lambda i, j: (i, 0, 0)),
            pl.BlockSpec((1, d), lambda i, j: (0, 0)),
            pl.BlockSpec((1, d), lambda i, j: (0, 0)),
        ],
        out_specs=pl.BlockSpec((1, d, bb), lambda i, j: (i, 0, j)),
        out_shape=jax.ShapeDtypeStruct((l, d, nb), jnp.float32),
    )


@functools.cache
def _sc_gather_call(nb: int, l: int, d: int, vocab: int, l0: int, nl: int):
    bw = nb // _NW
    assert bw * _NW == nb and bw % 128 == 0 and nl >= 4 and nl % 2 == 0
    ntab = vocab * l * d

    mesh = plsc.VectorSubcoreMesh(core_axis_name="c", subcore_axis_name="s")

    @functools.partial(
        pl.kernel,
        mesh=mesh,
        compiler_params=pltpu.CompilerParams(
            use_tc_tiling_on_sc=True, needs_layout_passes=False),
        out_type=jax.ShapeDtypeStruct((nl, d, nb), jnp.float32),
        scratch_types=[
            pltpu.VMEM((ntab,), jnp.float32),
            pltpu.VMEM((2, bw), jnp.int32),
            pltpu.VMEM((2, d, bw), jnp.float32),
            pltpu.SemaphoreType.DMA,
            [pltpu.SemaphoreType.DMA] * 2,
            [pltpu.SemaphoreType.DMA] * 2,
        ],
    )
    def gather(tab_hbm, xt_hbm, out_hbm, tab_v, x_v, blk_v,
               tab_sem, x_sems, o_sems):
        wid = lax.axis_index("s") * _NUM_CORES + lax.axis_index("c")
        b0 = wid * bw

        tab_cp = pltpu.async_copy(tab_hbm, tab_v, tab_sem)

        def x_fetch(li, s):
            return pltpu.async_copy(
                xt_hbm.at[pl.ds((l0 + li) * nb + b0, bw)],
                x_v.at[s], x_sems[s])

        def drain_x(s):
            pltpu.make_async_copy(
                xt_hbm.at[pl.ds(0, bw)], x_v.at[s], x_sems[s]).wait()

        def compute(li, s):
            off = (l0 + li) * d

            @plsc.parallel_loop(0, bw // 16, unroll=4)
            def _bg_body(bg):
                xv = x_v[s, pl.ds(bg * 16, 16)]
                base = xv * (l * d) + off
                for dd in range(d):
                    blk_v[s, dd, pl.ds(bg * 16, 16)] = plsc.load_gather(
                        tab_v, [base + dd])

        def out_write(li, s):
            return pltpu.async_copy(
                blk_v.at[s], out_hbm.at[li, :, pl.ds(b0, bw)], o_sems[s])

        def drain_out(s):
            pltpu.make_async_copy(
                blk_v.at[s], out_hbm.at[0, :, pl.ds(b0, bw)],
                o_sems[s]).wait()

        x_fetch(0, 0)
        x_fetch(1, 1)
        tab_cp.wait()
        for s in (0, 1):
            drain_x(s)
            compute(s, s)
            out_write(s, s)
            x_fetch(s + 2, s)

        def body2(i2, carry):
            for s in (0, 1):
                li = 2 + i2 * 2 + s
                drain_x(s)
                drain_out(s)
                compute(li, s)
                out_write(li, s)
                x_fetch(li + 2, s)
            return carry

        lax.fori_loop(0, (nl - 4) // 2, body2, None, unroll=False)
        for k in range(2):
            li = nl - 2 + k
            s = li % 2
            drain_x(s)
            drain_out(s)
            compute(li, s)
            out_write(li, s)
        drain_out(0)
        drain_out(1)

    return gather


def kernel(x, tok_embed, pos_embed, gamma, beta):
    nb, l = x.shape
    vocab, d = tok_embed.shape
    l_tc = l - _L_SC

    xt3 = x.T.reshape(l, 1, nb)
    tab = _table_call(tok_embed, pos_embed, gamma, beta).reshape(-1)
    sc_part = _sc_gather_call(nb, l, d, vocab, l_tc, _L_SC)(
        tab, x.T.reshape(-1))
    tc_part = _tc_select_call(nb, l, vocab, d, l_tc, 16384)(
        xt3, tok_embed, pos_embed.reshape(l, 1, d),
        gamma.reshape(1, d), beta.reshape(1, d))
    out_t = lax.dynamic_update_slice(tc_part, sc_part, (l_tc, 0, 0))
    return jnp.transpose(out_t, (2, 0, 1))


# speedup vs baseline: 10.4960x; 1.0457x over previous
import functools

import jax
import jax.numpy as jnp
from jax import lax
from jax.experimental import pallas as pl
from jax.experimental.pallas import tpu as pltpu
from jax.experimental.pallas import tpu_sc as plsc

_NUM_CORES = 2
_NUM_SUBCORES = 16
_NW = _NUM_CORES * _NUM_SUBCORES

_EPS = 1e-5
_L_SC = 8


def _ln_rows(tok_ref, pos, g, b):
    rows = []
    for v in range(tok_ref.shape[0]):
        e = pos + tok_ref[v:v + 1, :]
        m = jnp.mean(e, axis=1, keepdims=True)
        c = e - m
        var = jnp.mean(c * c, axis=1, keepdims=True)
        rows.append(c * lax.rsqrt(var + _EPS) * g + b)
    return rows


def _table_body(tok_ref, pos_ref, g_ref, b_ref, out_ref):
    out_ref[...] = jnp.concatenate(
        _ln_rows(tok_ref, pos_ref[...], g_ref[...], b_ref[...]), axis=0)


def _table_call(tok, pos, gamma, beta):
    vocab, d = tok.shape
    l = pos.shape[0]
    return pl.pallas_call(
        _table_body,
        out_shape=jax.ShapeDtypeStruct((vocab * l, d), jnp.float32),
    )(tok, pos, gamma.reshape(1, d), beta.reshape(1, d))


def _select_body(xt_ref, tok_ref, pos_ref, g_ref, b_ref, out_ref):
    d = tok_ref.shape[1]
    for k in range(out_ref.shape[0]):
        xt = xt_ref[k]
        acc = None
        for v, row in enumerate(
                _ln_rows(tok_ref, pos_ref[k], g_ref[...], b_ref[...])):
            col = row.reshape(d, 1)
            if acc is None:
                acc = jnp.broadcast_to(col, out_ref.shape[1:])
            else:
                acc = jnp.where(xt == v, col, acc)
        out_ref[k] = acc


@functools.cache
def _tc_select_call(nb: int, l: int, vocab: int, d: int, l_tc: int, bb: int,
                    lb: int):
    assert l_tc % lb == 0
    return pl.pallas_call(
        _select_body,
        grid=(l_tc // lb, nb // bb),
        in_specs=[
            pl.BlockSpec((lb, 1, bb), lambda i, j: (i, 0, j)),
            pl.BlockSpec((vocab, d), lambda i, j: (0, 0)),
            pl.BlockSpec((lb, 1, d), lambda i, j: (i, 0, 0)),
            pl.BlockSpec((1, d), lambda i, j: (0, 0)),
            pl.BlockSpec((1, d), lambda i, j: (0, 0)),
        ],
        out_specs=pl.BlockSpec((lb, d, bb), lambda i, j: (i, 0, j)),
        out_shape=jax.ShapeDtypeStruct((l, d, nb), jnp.float32),
    )


@functools.cache
def _sc_gather_call(nb: int, l: int, d: int, vocab: int, l0: int, nl: int):
    bw = nb // _NW
    assert bw * _NW == nb and bw % 128 == 0 and nl >= 4 and nl % 2 == 0
    ntab = vocab * l * d

    mesh = plsc.VectorSubcoreMesh(core_axis_name="c", subcore_axis_name="s")

    @functools.partial(
        pl.kernel,
        mesh=mesh,
        compiler_params=pltpu.CompilerParams(
            use_tc_tiling_on_sc=True, needs_layout_passes=False),
        out_type=jax.ShapeDtypeStruct((nl, d, nb), jnp.float32),
        scratch_types=[
            pltpu.VMEM((ntab,), jnp.float32),
            pltpu.VMEM((2, bw), jnp.int32),
            pltpu.VMEM((2, d, bw), jnp.float32),
            pltpu.SemaphoreType.DMA,
            [pltpu.SemaphoreType.DMA] * 2,
            [pltpu.SemaphoreType.DMA] * 2,
        ],
    )
    def gather(tab_hbm, xt_hbm, out_hbm, tab_v, x_v, blk_v,
               tab_sem, x_sems, o_sems):
        wid = lax.axis_index("s") * _NUM_CORES + lax.axis_index("c")
        b0 = wid * bw

        tab_cp = pltpu.async_copy(tab_hbm, tab_v, tab_sem)

        def x_fetch(li, s):
            return pltpu.async_copy(
                xt_hbm.at[pl.ds((l0 + li) * nb + b0, bw)],
                x_v.at[s], x_sems[s])

        def drain_x(s):
            pltpu.make_async_copy(
                xt_hbm.at[pl.ds(0, bw)], x_v.at[s], x_sems[s]).wait()

        def compute(li, s):
            off = (l0 + li) * d

            @plsc.parallel_loop(0, bw // 16, unroll=4)
            def _bg_body(bg):
                xv = x_v[s, pl.ds(bg * 16, 16)]
                base = xv * (l * d) + off
                for dd in range(d):
                    blk_v[s, dd, pl.ds(bg * 16, 16)] = plsc.load_gather(
                        tab_v, [base + dd])

        def out_write(li, s):
            return pltpu.async_copy(
                blk_v.at[s], out_hbm.at[li, :, pl.ds(b0, bw)], o_sems[s])

        def drain_out(s):
            pltpu.make_async_copy(
                blk_v.at[s], out_hbm.at[0, :, pl.ds(b0, bw)],
                o_sems[s]).wait()

        x_fetch(0, 0)
        x_fetch(1, 1)
        tab_cp.wait()
        for s in (0, 1):
            drain_x(s)
            compute(s, s)
            out_write(s, s)
            x_fetch(s + 2, s)

        def body2(i2, carry):
            for s in (0, 1):
                li = 2 + i2 * 2 + s
                drain_x(s)
                drain_out(s)
                compute(li, s)
                out_write(li, s)
                x_fetch(li + 2, s)
            return carry

        lax.fori_loop(0, (nl - 4) // 2, body2, None, unroll=False)
        for k in range(2):
            li = nl - 2 + k
            s = li % 2
            drain_x(s)
            drain_out(s)
            compute(li, s)
            out_write(li, s)
        drain_out(0)
        drain_out(1)

    return gather


def kernel(x, tok_embed, pos_embed, gamma, beta):
    nb, l = x.shape
    vocab, d = tok_embed.shape
    l_tc = l - _L_SC

    xt3 = x.T.reshape(l, 1, nb)
    tab = _table_call(tok_embed, pos_embed, gamma, beta).reshape(-1)
    sc_part = _sc_gather_call(nb, l, d, vocab, l_tc, _L_SC)(
        tab, x.T.reshape(-1))
    tc_part = _tc_select_call(nb, l, vocab, d, l_tc, 16384, 2)(
        xt3, tok_embed, pos_embed.reshape(l, 1, d),
        gamma.reshape(1, d), beta.reshape(1, d))
    out_t = lax.dynamic_update_slice(tc_part, sc_part, (l_tc, 0, 0))
    return jnp.transpose(out_t, (2, 0, 1))


# speedup vs baseline: 10.6607x; 1.0157x over previous
import functools

import jax
import jax.numpy as jnp
from jax import lax
from jax.experimental import pallas as pl
from jax.experimental.pallas import tpu as pltpu
from jax.experimental.pallas import tpu_sc as plsc

_NUM_CORES = 2
_NUM_SUBCORES = 16
_NW = _NUM_CORES * _NUM_SUBCORES

_EPS = 1e-5
_L_SC = 8


def _ln_rows(tok_ref, pos, g, b):
    rows = []
    for v in range(tok_ref.shape[0]):
        e = pos + tok_ref[v:v + 1, :]
        m = jnp.mean(e, axis=1, keepdims=True)
        c = e - m
        var = jnp.mean(c * c, axis=1, keepdims=True)
        rows.append(c * lax.rsqrt(var + _EPS) * g + b)
    return rows


def _table_body(tok_ref, pos_ref, g_ref, b_ref, out_ref):
    out_ref[...] = jnp.concatenate(
        _ln_rows(tok_ref, pos_ref[...], g_ref[...], b_ref[...]), axis=0)


def _table_call(tok, pos, gamma, beta):
    vocab, d = tok.shape
    l = pos.shape[0]
    return pl.pallas_call(
        _table_body,
        out_shape=jax.ShapeDtypeStruct((vocab * l, d), jnp.float32),
    )(tok, pos, gamma.reshape(1, d), beta.reshape(1, d))


def _select_body(xt_ref, tok_ref, pos_ref, g_ref, b_ref, out_ref):
    d = tok_ref.shape[1]
    for k in range(out_ref.shape[0]):
        xt = xt_ref[k]
        acc = None
        for v, row in enumerate(
                _ln_rows(tok_ref, pos_ref[k], g_ref[...], b_ref[...])):
            col = row.reshape(d, 1)
            if acc is None:
                acc = jnp.broadcast_to(col, out_ref.shape[1:])
            else:
                acc = jnp.where(xt == v, col, acc)
        out_ref[k] = acc


@functools.cache
def _tc_select_call(nb: int, l: int, vocab: int, d: int, l_tc: int, bb: int,
                    lb: int):
    assert l_tc % lb == 0
    return pl.pallas_call(
        _select_body,
        grid=(l_tc // lb, nb // bb),
        in_specs=[
            pl.BlockSpec((lb, 1, bb), lambda i, j: (i, 0, j)),
            pl.BlockSpec((vocab, d), lambda i, j: (0, 0)),
            pl.BlockSpec((lb, 1, d), lambda i, j: (i, 0, 0)),
            pl.BlockSpec((1, d), lambda i, j: (0, 0)),
            pl.BlockSpec((1, d), lambda i, j: (0, 0)),
        ],
        out_specs=pl.BlockSpec((lb, d, bb), lambda i, j: (i, 0, j)),
        out_shape=jax.ShapeDtypeStruct((l, d, nb), jnp.float32),
    )


@functools.cache
def _sc_gather_call(nb: int, l: int, d: int, vocab: int, l0: int, nl: int):
    bw = nb // _NW
    assert bw * _NW == nb and bw % 128 == 0 and nl >= 4 and nl % 2 == 0
    ntab = vocab * l * d

    mesh = plsc.VectorSubcoreMesh(core_axis_name="c", subcore_axis_name="s")

    @functools.partial(
        pl.kernel,
        mesh=mesh,
        compiler_params=pltpu.CompilerParams(
            use_tc_tiling_on_sc=True, needs_layout_passes=False),
        out_type=jax.ShapeDtypeStruct((nl, d, nb), jnp.float32),
        scratch_types=[
            pltpu.VMEM((ntab,), jnp.float32),
            pltpu.VMEM((2, bw), jnp.int32),
            pltpu.VMEM((2, d, bw), jnp.float32),
            pltpu.SemaphoreType.DMA,
            [pltpu.SemaphoreType.DMA] * 2,
            [pltpu.SemaphoreType.DMA] * 2,
        ],
    )
    def gather(tab_hbm, xt_hbm, out_hbm, tab_v, x_v, blk_v,
               tab_sem, x_sems, o_sems):
        wid = lax.axis_index("s") * _NUM_CORES + lax.axis_index("c")
        b0 = wid * bw

        tab_cp = pltpu.async_copy(tab_hbm, tab_v, tab_sem)

        def x_fetch(li, s):
            return pltpu.async_copy(
                xt_hbm.at[pl.ds((l0 + li) * nb + b0, bw)],
                x_v.at[s], x_sems[s])

        def drain_x(s):
            pltpu.make_async_copy(
                xt_hbm.at[pl.ds(0, bw)], x_v.at[s], x_sems[s]).wait()

        def compute(li, s):
            off = (l0 + li) * d

            @plsc.parallel_loop(0, bw // 16, unroll=4)
            def _bg_body(bg):
                xv = x_v[s, pl.ds(bg * 16, 16)]
                base = xv * (l * d) + off
                for dd in range(d):
                    blk_v[s, dd, pl.ds(bg * 16, 16)] = plsc.load_gather(
                        tab_v, [base + dd])

        def out_write(li, s):
            return pltpu.async_copy(
                blk_v.at[s], out_hbm.at[li, :, pl.ds(b0, bw)], o_sems[s])

        def drain_out(s):
            pltpu.make_async_copy(
                blk_v.at[s], out_hbm.at[0, :, pl.ds(b0, bw)],
                o_sems[s]).wait()

        x_fetch(0, 0)
        x_fetch(1, 1)
        tab_cp.wait()
        for s in (0, 1):
            drain_x(s)
            compute(s, s)
            out_write(s, s)
            x_fetch(s + 2, s)

        def body2(i2, carry):
            for s in (0, 1):
                li = 2 + i2 * 2 + s
                drain_x(s)
                drain_out(s)
                compute(li, s)
                out_write(li, s)
                x_fetch(li + 2, s)
            return carry

        lax.fori_loop(0, (nl - 4) // 2, body2, None, unroll=False)
        for k in range(2):
            li = nl - 2 + k
            s = li % 2
            drain_x(s)
            drain_out(s)
            compute(li, s)
            out_write(li, s)
        drain_out(0)
        drain_out(1)

    return gather


def kernel(x, tok_embed, pos_embed, gamma, beta):
    nb, l = x.shape
    vocab, d = tok_embed.shape
    l_tc = l - _L_SC

    xt3 = x.T.reshape(l, 1, nb)
    tab = _table_call(tok_embed, pos_embed, gamma, beta).reshape(-1)
    sc_part = _sc_gather_call(nb, l, d, vocab, l_tc, _L_SC)(
        tab, x.T.reshape(-1))
    tc_part = _tc_select_call(nb, l, vocab, d, l_tc, 16384, 4)(
        xt3, tok_embed, pos_embed.reshape(l, 1, d),
        gamma.reshape(1, d), beta.reshape(1, d))
    out_t = lax.dynamic_update_slice(tc_part, sc_part, (l_tc, 0, 0))
    return jnp.transpose(out_t, (2, 0, 1))
